# baseline (speedup 1.0000x reference)
import functools
import math

import jax
import jax.numpy as jnp
from jax import lax
from jax.experimental import pallas as pl
from jax.experimental.pallas import tpu as pltpu
from jax.experimental.pallas import tpu_sc as plsc

N = 768
_REQ = int((1 - 0.1) * N * N)
_K = math.ceil(_REQ / N)
_BT = 512

_NC = 2
_NS = 16
_NW = _NC * _NS
_L = 16
_RPW = N // _NW
_NCH = N // _L


@functools.partial(
    pl.kernel,
    out_type=jax.ShapeDtypeStruct((N, N), jnp.float32),
    mesh=plsc.VectorSubcoreMesh(core_axis_name="c", subcore_axis_name="s"),
    compiler_params=pltpu.CompilerParams(needs_layout_passes=False),
    scratch_types=[
        pltpu.VMEM((N,), jnp.float32),
        pltpu.VMEM((N,), jnp.float32),
        pltpu.VMEM((_RPW, N), jnp.float32),
        pltpu.VMEM((_RPW, N), jnp.float32),
        pltpu.VMEM((_L,), jnp.float32),
    ],
)
def _build_wt(vt_hbm, alpha_hbm, wt_hbm, a_v, e_v, f_v, o_v, t_v):
    wid = lax.axis_index("s") * _NC + lax.axis_index("c")
    base = wid * _RPW
    pltpu.sync_copy(alpha_hbm, a_v)
    pltpu.sync_copy(vt_hbm.at[pl.ds(base, _RPW)], f_v)

    lane = lax.iota(jnp.int32, _L)

    def _xlane(vec, op):
        for shift in (8, 4, 2, 1):
            t_v[pl.ds(0, _L)] = vec
            vec = op(vec, plsc.load_gather(t_v, [lax.bitwise_xor(lane, shift)]))
        return vec

    def _mx(i, m):
        return jnp.maximum(m, a_v[pl.ds(i * _L, _L)])

    m = _xlane(lax.fori_loop(0, _NCH, _mx, jnp.full((_L,), -jnp.inf, jnp.float32)),
               jnp.maximum)

    def _ex(i, s):
        v = jnp.exp(a_v[pl.ds(i * _L, _L)] - m)
        e_v[pl.ds(i * _L, _L)] = v
        return s + v

    s = _xlane(lax.fori_loop(0, _NCH, _ex, jnp.zeros((_L,), jnp.float32)),
               jnp.add)
    kscale = _K / s

    def _gate(i, t):
        v = e_v[pl.ds(i * _L, _L)] * kscale
        e_v[pl.ds(i * _L, _L)] = jnp.clip(v, 0.0, 1.0)
        return t

    lax.fori_loop(0, _NCH, _gate, 0)

    def _row(j, t):
        c = base + j
        jsplat = jnp.full((_L,), 0, jnp.int32) + j
        gate = plsc.load_gather(e_v, [jsplat * 0 + c])

        def _chunk(jj, u):
            col = lax.rem(jj * _L + lane + (N - c), N)
            v = plsc.load_gather(f_v, [jsplat, col])
            plsc.store_scatter(o_v, [jsplat, jj * _L + lane], v * gate)
            return u

        lax.fori_loop(0, _NCH, _chunk, 0)
        return t

    lax.fori_loop(0, _RPW, _row, 0)
    pltpu.sync_copy(o_v, wt_hbm.at[pl.ds(base, _RPW)])


def _mm_kernel(wt_ref, x_ref, out_ref):
    out_ref[...] = jnp.dot(x_ref[...], wt_ref[...],
                           preferred_element_type=jnp.float32)


@jax.jit
def kernel(x, V, alpha):
    wt = _build_wt(V.T, alpha)
    batch = x.shape[0]
    return pl.pallas_call(
        _mm_kernel,
        grid=(batch // _BT,),
        in_specs=[
            pl.BlockSpec((N, N), lambda i: (0, 0)),
            pl.BlockSpec((_BT, N), lambda i: (i, 0)),
        ],
        out_specs=pl.BlockSpec((_BT, N), lambda i: (i, 0)),
        out_shape=jax.ShapeDtypeStruct((batch, N), jnp.float32),
    )(wt, x)


# speedup vs baseline: 1.4464x; 1.4464x over previous
import functools
import math

import jax
import jax.numpy as jnp
from jax import lax
from jax.experimental import pallas as pl
from jax.experimental.pallas import tpu as pltpu
from jax.experimental.pallas import tpu_sc as plsc

N = 768
_REQ = int((1 - 0.1) * N * N)
_K = math.ceil(_REQ / N)
_BT = 512

_NC = 2
_NS = 16
_NW = _NC * _NS
_L = 16
_RPW = N // _NW
_NCH = N // _L


@functools.partial(
    pl.kernel,
    out_type=jax.ShapeDtypeStruct((N, N), jnp.float32),
    mesh=plsc.VectorSubcoreMesh(core_axis_name="c", subcore_axis_name="s"),
    compiler_params=pltpu.CompilerParams(needs_layout_passes=False),
    scratch_types=[
        pltpu.VMEM((N,), jnp.float32),
        pltpu.VMEM((N,), jnp.float32),
        pltpu.VMEM((_RPW, N), jnp.float32),
        pltpu.VMEM((_RPW, N), jnp.float32),
        pltpu.VMEM((_L,), jnp.float32),
    ],
)
def _build_wt(vt_hbm, alpha_hbm, wt_hbm, a_v, e_v, f_v, o_v, t_v):
    wid = lax.axis_index("s") * _NC + lax.axis_index("c")
    base = wid * _RPW
    pltpu.sync_copy(alpha_hbm, a_v)
    pltpu.sync_copy(vt_hbm.at[pl.ds(base, _RPW)], f_v)

    lane = lax.iota(jnp.int32, _L)

    def _xlane(vec, op):
        for shift in (8, 4, 2, 1):
            t_v[pl.ds(0, _L)] = vec
            vec = op(vec, plsc.load_gather(t_v, [lax.bitwise_xor(lane, shift)]))
        return vec

    m = jnp.full((_L,), -jnp.inf, jnp.float32)
    for i in range(_NCH):
        m = jnp.maximum(m, a_v[pl.ds(i * _L, _L)])
    m = _xlane(m, jnp.maximum)

    s = jnp.zeros((_L,), jnp.float32)
    for i in range(_NCH):
        v = jnp.exp(a_v[pl.ds(i * _L, _L)] - m)
        e_v[pl.ds(i * _L, _L)] = v
        s = s + v
    s = _xlane(s, jnp.add)
    kscale = _K / s

    for i in range(_NCH):
        v = e_v[pl.ds(i * _L, _L)] * kscale
        e_v[pl.ds(i * _L, _L)] = jnp.clip(v, 0.0, 1.0)

    def _row(j, t):
        c = base + j
        jsplat = lane * 0 + j
        gate = plsc.load_gather(e_v, [lane * 0 + c])
        vb = lane + (N - c)
        for jj in range(_NCH):
            idx = vb + (jj * _L)
            idx = jnp.where(idx >= N, idx - N, idx)
            v = plsc.load_gather(f_v, [jsplat, idx])
            o_v[j, pl.ds(jj * _L, _L)] = v * gate
        return t

    lax.fori_loop(0, _RPW, _row, 0)
    pltpu.sync_copy(o_v, wt_hbm.at[pl.ds(base, _RPW)])


def _mm_kernel(wt_ref, x_ref, out_ref):
    out_ref[...] = jnp.dot(x_ref[...], wt_ref[...],
                           preferred_element_type=jnp.float32)


@jax.jit
def kernel(x, V, alpha):
    wt = _build_wt(V.T, alpha)
    batch = x.shape[0]
    return pl.pallas_call(
        _mm_kernel,
        grid=(batch // _BT,),
        in_specs=[
            pl.BlockSpec((N, N), lambda i: (0, 0)),
            pl.BlockSpec((_BT, N), lambda i: (i, 0)),
        ],
        out_specs=pl.BlockSpec((_BT, N), lambda i: (i, 0)),
        out_shape=jax.ShapeDtypeStruct((batch, N), jnp.float32),
    )(wt, x)


# speedup vs baseline: 3.1352x; 2.1676x over previous
import math

import jax
import jax.numpy as jnp
from jax.experimental import pallas as pl
from jax.experimental.pallas import tpu as pltpu

N = 768
_REQ = int((1 - 0.1) * N * N)
_K = math.ceil(_REQ / N)
_BT = 512


def _fc_kernel(a_ref, vt_ref, x_ref, out_ref, wt_ref):
    @pl.when(pl.program_id(0) == 0)
    def _build_wt():
        a = a_ref[...]
        e = jnp.exp(a - jnp.max(a))
        atk = jnp.clip((_K / jnp.sum(e)) * e, 0.0, 1.0)
        w = (vt_ref[...] * atk).astype(jnp.bfloat16)
        row = jax.lax.broadcasted_iota(jnp.int32, (N, 1), 0)
        for b in range(10):
            amt = 1 << b
            rolled = jnp.concatenate([w[:, N - amt:], w[:, :N - amt]], axis=1)
            w = jnp.where((row & amt) != 0, rolled, w)
        wt_ref[...] = w

    out_ref[...] = jnp.dot(x_ref[...].astype(jnp.bfloat16), wt_ref[...],
                           preferred_element_type=jnp.float32)


@jax.jit
def kernel(x, V, alpha):
    batch = x.shape[0]
    return pl.pallas_call(
        _fc_kernel,
        grid=(batch // _BT,),
        in_specs=[
            pl.BlockSpec((1, N), lambda i: (0, 0)),
            pl.BlockSpec((N, N), lambda i: (0, 0)),
            pl.BlockSpec((_BT, N), lambda i: (i, 0)),
        ],
        out_specs=pl.BlockSpec((_BT, N), lambda i: (i, 0)),
        out_shape=jax.ShapeDtypeStruct((batch, N), jnp.float32),
        scratch_shapes=[pltpu.VMEM((N, N), jnp.bfloat16)],
    )(alpha.reshape(1, N), V.T, x)


# speedup vs baseline: 3.4947x; 1.1147x over previous
import math

import jax
import jax.numpy as jnp
from jax.experimental import pallas as pl
from jax.experimental.pallas import tpu as pltpu

N = 768
_REQ = int((1 - 0.1) * N * N)
_K = math.ceil(_REQ / N)
_BT = 512


def _fc_kernel(a_ref, v_ref, x_ref, out_ref, wt_ref):
    @pl.when(pl.program_id(0) == 0)
    def _build_wt():
        a = a_ref[...]
        e = jnp.exp(a - jnp.max(a))
        atk = jnp.clip((_K / jnp.sum(e)) * e, 0.0, 1.0)
        w = jnp.transpose((v_ref[...] * atk).astype(jnp.bfloat16))
        row = jax.lax.broadcasted_iota(jnp.int32, (N, 1), 0)
        for b in range(10):
            amt = 1 << b
            rolled = jnp.concatenate([w[:, N - amt:], w[:, :N - amt]], axis=1)
            w = jnp.where((row & amt) != 0, rolled, w)
        wt_ref[...] = w

    out_ref[...] = jnp.dot(x_ref[...].astype(jnp.bfloat16), wt_ref[...],
                           preferred_element_type=jnp.float32)


@jax.jit
def kernel(x, V, alpha):
    batch = x.shape[0]
    return pl.pallas_call(
        _fc_kernel,
        grid=(batch // _BT,),
        in_specs=[
            pl.BlockSpec((N, 1), lambda i: (0, 0)),
            pl.BlockSpec((N, N), lambda i: (0, 0)),
            pl.BlockSpec((_BT, N), lambda i: (i, 0)),
        ],
        out_specs=pl.BlockSpec((_BT, N), lambda i: (i, 0)),
        out_shape=jax.ShapeDtypeStruct((batch, N), jnp.float32),
        scratch_shapes=[pltpu.VMEM((N, N), jnp.bfloat16)],
    )(alpha.reshape(N, 1), V, x)
